# baseline (speedup 1.0000x reference)
import functools

import jax
import jax.numpy as jnp
from jax import lax
from jax.experimental import pallas as pl
from jax.experimental.pallas import tpu as pltpu
from jax.experimental.pallas import tpu_sc as plsc

_KB = 1024
_NB_SC = 4
_CH = 32
_TPT = 256


def _weights_body(len_ref, c0_ref, c1_ref, w_ref):
    L = c0_ref.shape[-1]
    b = pl.program_id(0)
    n = len_ref[b]
    c0 = c0_ref[0]
    c1 = c1_ref[0]
    pos = jax.lax.broadcasted_iota(jnp.int32, (1, L), 1)
    valid = pos < n

    p0 = jnp.roll(c0, 1, axis=1)
    p1 = jnp.roll(c1, 1, axis=1)
    same = (c0 == p0) & (c1 == p1)
    nb = ((pos == 0) | jnp.logical_not(same)) & valid

    s = jnp.where(nb, pos, -1)
    k = 1
    while k < L:
        sh = jnp.where(pos >= k, jnp.roll(s, k, axis=1), -1)
        s = jnp.maximum(s, sh)
        k *= 2

    big = jnp.int32(2**30)
    t = jnp.where(nb, pos, big)
    k = 1
    while k < L:
        sh = jnp.where(pos < L - k, jnp.roll(t, -k, axis=1), big)
        t = jnp.minimum(t, sh)
        k *= 2
    ns = jnp.where(pos < L - 1, jnp.roll(t, -1, axis=1), big)
    ns = jnp.minimum(ns, n)

    run_len = (ns - s).astype(jnp.float32)
    num_groups = jnp.sum(nb.astype(jnp.float32))
    denom = num_groups * run_len
    safe = valid & (denom > 0)
    w_ref[0] = jnp.where(safe, 1.0 / jnp.where(denom > 0, denom, 1.0), 0.0)


def _matmul_body(w_ref, feat_ref, out_ref):
    nb_tc = out_ref.shape[0]
    bq = pl.program_id(0)
    kb = pl.program_id(1)

    @pl.when((bq == 0) & (kb == 0))
    def _():
        out_ref[...] = jnp.zeros_like(out_ref)

    w_chunk = w_ref[0]
    row = jax.lax.broadcasted_iota(jnp.int32, (nb_tc, w_chunk.shape[-1]), 0)
    lhs = jnp.where(row == bq, jnp.broadcast_to(w_chunk, row.shape), 0.0)
    f = feat_ref[0, 0]
    out_ref[...] += jnp.dot(lhs, f, preferred_element_type=jnp.float32)


def _sc_body(feat_hbm, w_hbm, out_hbm, buf0, buf1, wbuf, acc, red, shared,
             sem0, sem1):
    D = 1024
    L = 2048
    N = 4
    c = lax.axis_index("c")
    s = lax.axis_index("s")
    r = 2 * c + s // 8
    t0 = (s % 8) * _TPT
    feat_base = ((r * N + N - 1) * L + t0) * D
    feat_base = pl.multiple_of(feat_base, _CH * D)
    w_base = pl.multiple_of(r * L + t0, _TPT)

    pltpu.sync_copy(w_hbm.at[pl.ds(w_base, _TPT)], wbuf)
    for db in range(D // 16):
        acc[pl.ds(db * 16, 16)] = jnp.zeros((16,), jnp.float32)

    bufs = (buf0, buf1)
    sems = (sem0, sem1)
    nch = _TPT // _CH
    handles = [None, None]
    handles[0] = pltpu.async_copy(
        feat_hbm.at[pl.ds(feat_base, _CH * D)], buf0, sem0)

    for ci in range(nch):
        cur = ci % 2
        if ci + 1 < nch:
            nxt = (ci + 1) % 2
            src = pl.multiple_of(feat_base + (ci + 1) * _CH * D, _CH * D)
            handles[nxt] = pltpu.async_copy(
                feat_hbm.at[pl.ds(src, _CH * D)], bufs[nxt], sems[nxt])
        handles[cur].wait()
        cbuf = bufs[cur]

        def tok_body(i, carry, cbuf=cbuf, ci=ci):
            grp = pl.multiple_of((ci * _CH + i) // 16 * 16, 16)
            wv16 = wbuf[pl.ds(grp, 16)]
            lane = jnp.zeros((16,), jnp.int32) + i % 16
            wv = lax.gather(
                wv16, lane[:, None],
                lax.GatherDimensionNumbers(
                    offset_dims=(), collapsed_slice_dims=(0,),
                    start_index_map=(0,)),
                slice_sizes=(1,),
                mode=lax.GatherScatterMode.PROMISE_IN_BOUNDS)
            off = i * D
            for db in range(D // 16):
                x = cbuf[pl.ds(off + db * 16, 16)]
                plsc.addupdate(acc.at[pl.ds(db * 16, 16)], wv * x)
            return carry

        lax.fori_loop(0, _CH, tok_body, 0)

    pltpu.sync_copy(acc, shared.at[pl.ds(pl.multiple_of(s * D, D), D)])
    plsc.subcore_barrier()

    @pl.when(s % 8 == 0)
    def _():
        rbase = pl.multiple_of(s * D, D)
        pltpu.sync_copy(shared.at[pl.ds(rbase, 8 * D)], red)
        for db in range(D // 16):
            tot = red[pl.ds(db * 16, 16)]
            for j in range(1, 8):
                tot = tot + red[pl.ds(j * D + db * 16, 16)]
            acc[pl.ds(db * 16, 16)] = tot
        pltpu.sync_copy(acc, out_hbm.at[pl.ds(pl.multiple_of(r * D, D), D)])


def kernel(input_feature, input_lengths, vq_indices):
    B, N, L, D = input_feature.shape
    c0 = vq_indices[:, :, 0].reshape(B, 1, L).astype(jnp.int32)
    c1 = vq_indices[:, :, 1].reshape(B, 1, L).astype(jnp.int32)
    lengths = input_lengths.astype(jnp.int32)

    w = pl.pallas_call(
        _weights_body,
        grid=(B,),
        in_specs=[
            pl.BlockSpec(memory_space=pltpu.SMEM),
            pl.BlockSpec((1, 1, L), lambda b: (b, 0, 0)),
            pl.BlockSpec((1, 1, L), lambda b: (b, 0, 0)),
        ],
        out_specs=pl.BlockSpec((1, 1, L), lambda b: (b, 0, 0)),
        out_shape=jax.ShapeDtypeStruct((B, 1, L), jnp.float32),
    )(lengths, c0, c1)

    feat1d = input_feature.reshape(B * N * L * D)
    w_flat = w.reshape(B * L)
    mesh = plsc.VectorSubcoreMesh(core_axis_name="c", subcore_axis_name="s")
    sc_call = functools.partial(
        pl.kernel,
        mesh=mesh,
        out_type=jax.ShapeDtypeStruct((_NB_SC * D,), jnp.float32),
        scratch_types=[
            pltpu.VMEM((_CH * D,), jnp.float32),
            pltpu.VMEM((_CH * D,), jnp.float32),
            pltpu.VMEM((_TPT,), jnp.float32),
            pltpu.VMEM((D,), jnp.float32),
            pltpu.VMEM((8 * D,), jnp.float32),
            pltpu.VMEM_SHARED((16 * D,), jnp.float32),
            pltpu.SemaphoreType.DMA,
            pltpu.SemaphoreType.DMA,
        ],
    )(_sc_body)
    out_sc = sc_call(feat1d, w_flat).reshape(_NB_SC, D)

    nb_tc = B - _NB_SC
    nk = L // _KB
    out_tc = pl.pallas_call(
        _matmul_body,
        grid=(nb_tc, nk),
        in_specs=[
            pl.BlockSpec((1, 1, _KB), lambda bq, kb: (_NB_SC + bq, 0, kb)),
            pl.BlockSpec((1, 1, _KB, D), lambda bq, kb: (_NB_SC + bq, N - 1, kb, 0)),
        ],
        out_specs=pl.BlockSpec((nb_tc, D), lambda bq, kb: (0, 0)),
        out_shape=jax.ShapeDtypeStruct((nb_tc, D), jnp.float32),
    )(w, input_feature)

    return jnp.concatenate([out_sc, out_tc], axis=0)


# speedup vs baseline: 5.3927x; 5.3927x over previous
import functools

import jax
import jax.numpy as jnp
from jax import lax
from jax.experimental import pallas as pl
from jax.experimental.pallas import tpu as pltpu
from jax.experimental.pallas import tpu_sc as plsc

_KB = 1024
_NB_SC = 4
_CH = 32
_TPT = 256


def _weights_body(len_ref, c0_ref, c1_ref, w_ref):
    L = c0_ref.shape[-1]
    b = pl.program_id(0)
    n = len_ref[b]
    c0 = c0_ref[0]
    c1 = c1_ref[0]
    pos = jax.lax.broadcasted_iota(jnp.int32, (1, L), 1)
    valid = pos < n

    p0 = jnp.roll(c0, 1, axis=1)
    p1 = jnp.roll(c1, 1, axis=1)
    same = (c0 == p0) & (c1 == p1)
    nb = ((pos == 0) | jnp.logical_not(same)) & valid

    s = jnp.where(nb, pos, -1)
    k = 1
    while k < L:
        sh = jnp.where(pos >= k, jnp.roll(s, k, axis=1), -1)
        s = jnp.maximum(s, sh)
        k *= 2

    big = jnp.int32(2**30)
    t = jnp.where(nb, pos, big)
    k = 1
    while k < L:
        sh = jnp.where(pos < L - k, jnp.roll(t, -k, axis=1), big)
        t = jnp.minimum(t, sh)
        k *= 2
    ns = jnp.where(pos < L - 1, jnp.roll(t, -1, axis=1), big)
    ns = jnp.minimum(ns, n)

    run_len = (ns - s).astype(jnp.float32)
    num_groups = jnp.sum(nb.astype(jnp.float32))
    denom = num_groups * run_len
    safe = valid & (denom > 0)
    w_ref[0] = jnp.where(safe, 1.0 / jnp.where(denom > 0, denom, 1.0), 0.0)


def _matmul_body(w_ref, feat_ref, out_ref):
    nb_tc = out_ref.shape[0]
    bq = pl.program_id(0)
    kb = pl.program_id(1)

    @pl.when((bq == 0) & (kb == 0))
    def _():
        out_ref[...] = jnp.zeros_like(out_ref)

    w_chunk = w_ref[0]
    row = jax.lax.broadcasted_iota(jnp.int32, (nb_tc, w_chunk.shape[-1]), 0)
    lhs = jnp.where(row == bq, jnp.broadcast_to(w_chunk, row.shape), 0.0)
    f = feat_ref[0, 0]
    out_ref[...] += jnp.dot(lhs, f, preferred_element_type=jnp.float32)


def _splat(wbuf, tok16, lane):
    wv16 = wbuf[pl.ds(tok16, 16)]
    return lax.gather(
        wv16, lane[:, None],
        lax.GatherDimensionNumbers(
            offset_dims=(), collapsed_slice_dims=(0,), start_index_map=(0,)),
        slice_sizes=(1,),
        mode=lax.GatherScatterMode.PROMISE_IN_BOUNDS)


def _sc_body(feat_hbm, w_hbm, out_hbm, buf0, buf1, wbuf, acc, red, shared,
             sem0, sem1):
    D = 1024
    N = feat_hbm.shape[1]
    c = lax.axis_index("c")
    s = lax.axis_index("s")
    r = 2 * c + s // 8
    t0 = (s % 8) * _TPT

    pltpu.sync_copy(w_hbm.at[r, 0, pl.ds(pl.multiple_of(t0, _TPT), _TPT)],
                    wbuf)

    bufs = (buf0, buf1)
    sems = (sem0, sem1)
    nch = _TPT // _CH
    handles = [None, None]

    def issue(ci, which):
        tok = pl.multiple_of(t0 + ci * _CH, _CH)
        return pltpu.async_copy(
            feat_hbm.at[r, N - 1, pl.ds(tok, _CH), :], bufs[which],
            sems[which])

    handles[0] = issue(0, 0)

    for ci in range(nch):
        cur = ci % 2
        if ci + 1 < nch:
            handles[(ci + 1) % 2] = issue(ci + 1, (ci + 1) % 2)
        handles[cur].wait()
        cbuf = bufs[cur]

        for g in range(4):
            col0 = g * 256
            if ci == 0:
                init = tuple(jnp.zeros((16,), jnp.float32)
                             for _ in range(16))
            else:
                init = tuple(acc[pl.ds(col0 + j * 16, 16)]
                             for j in range(16))

            def tok_body(i, accs, cbuf=cbuf, ci=ci, col0=col0):
                grp = pl.multiple_of((ci * _CH + i) // 16 * 16, 16)
                lane = jnp.zeros((16,), jnp.int32) + i % 16
                wv = _splat(wbuf, grp, lane)
                return tuple(
                    a + wv * cbuf[i, pl.ds(col0 + j * 16, 16)]
                    for j, a in enumerate(accs))

            accs = lax.fori_loop(0, _CH, tok_body, init)
            for j in range(16):
                acc[pl.ds(col0 + j * 16, 16)] = accs[j]

    pltpu.sync_copy(acc, shared.at[pl.ds(pl.multiple_of(s * D, D), D)])
    plsc.subcore_barrier()

    @pl.when(s % 8 == 0)
    def _():
        rbase = pl.multiple_of(s * D, D)
        pltpu.sync_copy(shared.at[pl.ds(rbase, 8 * D)], red)
        for db in range(D // 16):
            tot = red[pl.ds(db * 16, 16)]
            for j in range(1, 8):
                tot = tot + red[pl.ds(j * D + db * 16, 16)]
            acc[pl.ds(db * 16, 16)] = tot
        pltpu.sync_copy(acc, out_hbm.at[pl.ds(pl.multiple_of(r * D, D), D)])


def kernel(input_feature, input_lengths, vq_indices):
    B, N, L, D = input_feature.shape
    c0 = vq_indices[:, :, 0].reshape(B, 1, L).astype(jnp.int32)
    c1 = vq_indices[:, :, 1].reshape(B, 1, L).astype(jnp.int32)
    lengths = input_lengths.astype(jnp.int32)

    w = pl.pallas_call(
        _weights_body,
        grid=(B,),
        in_specs=[
            pl.BlockSpec(memory_space=pltpu.SMEM),
            pl.BlockSpec((1, 1, L), lambda b: (b, 0, 0)),
            pl.BlockSpec((1, 1, L), lambda b: (b, 0, 0)),
        ],
        out_specs=pl.BlockSpec((1, 1, L), lambda b: (b, 0, 0)),
        out_shape=jax.ShapeDtypeStruct((B, 1, L), jnp.float32),
    )(lengths, c0, c1)

    mesh = plsc.VectorSubcoreMesh(core_axis_name="c", subcore_axis_name="s")
    sc_call = functools.partial(
        pl.kernel,
        mesh=mesh,
        out_type=jax.ShapeDtypeStruct((_NB_SC * D,), jnp.float32),
        scratch_types=[
            pltpu.VMEM((_CH, D), jnp.float32),
            pltpu.VMEM((_CH, D), jnp.float32),
            pltpu.VMEM((_TPT,), jnp.float32),
            pltpu.VMEM((D,), jnp.float32),
            pltpu.VMEM((8 * D,), jnp.float32),
            pltpu.VMEM_SHARED((16 * D,), jnp.float32),
            pltpu.SemaphoreType.DMA,
            pltpu.SemaphoreType.DMA,
        ],
    )(_sc_body)
    out_sc = sc_call(input_feature, w).reshape(_NB_SC, D)

    nb_tc = B - _NB_SC
    nk = L // _KB
    out_tc = pl.pallas_call(
        _matmul_body,
        grid=(nb_tc, nk),
        in_specs=[
            pl.BlockSpec((1, 1, _KB), lambda bq, kb: (_NB_SC + bq, 0, kb)),
            pl.BlockSpec((1, 1, _KB, D), lambda bq, kb: (_NB_SC + bq, N - 1, kb, 0)),
        ],
        out_specs=pl.BlockSpec((nb_tc, D), lambda bq, kb: (0, 0)),
        out_shape=jax.ShapeDtypeStruct((nb_tc, D), jnp.float32),
    )(w, input_feature)

    return jnp.concatenate([out_sc, out_tc], axis=0)


# speedup vs baseline: 5.7760x; 1.0711x over previous
import functools

import jax
import jax.numpy as jnp
from jax import lax
from jax.experimental import pallas as pl
from jax.experimental.pallas import tpu as pltpu
from jax.experimental.pallas import tpu_sc as plsc

_KB = 1024
_NB_SC = 1
_CH = 32
_TPT = 64


def _weights_body(len_ref, c0_ref, c1_ref, w_ref):
    L = c0_ref.shape[-1]
    b = pl.program_id(0)
    n = len_ref[b]
    c0 = c0_ref[0]
    c1 = c1_ref[0]
    pos = jax.lax.broadcasted_iota(jnp.int32, (1, L), 1)
    valid = pos < n

    p0 = jnp.roll(c0, 1, axis=1)
    p1 = jnp.roll(c1, 1, axis=1)
    same = (c0 == p0) & (c1 == p1)
    nb = ((pos == 0) | jnp.logical_not(same)) & valid

    s = jnp.where(nb, pos, -1)
    k = 1
    while k < L:
        sh = jnp.where(pos >= k, jnp.roll(s, k, axis=1), -1)
        s = jnp.maximum(s, sh)
        k *= 2

    big = jnp.int32(2**30)
    t = jnp.where(nb, pos, big)
    k = 1
    while k < L:
        sh = jnp.where(pos < L - k, jnp.roll(t, -k, axis=1), big)
        t = jnp.minimum(t, sh)
        k *= 2
    ns = jnp.where(pos < L - 1, jnp.roll(t, -1, axis=1), big)
    ns = jnp.minimum(ns, n)

    run_len = (ns - s).astype(jnp.float32)
    num_groups = jnp.sum(nb.astype(jnp.float32))
    denom = num_groups * run_len
    safe = valid & (denom > 0)
    w_ref[0] = jnp.where(safe, 1.0 / jnp.where(denom > 0, denom, 1.0), 0.0)


def _matmul_body(w_ref, feat_ref, out_ref):
    nb_tc = out_ref.shape[0]
    bq = pl.program_id(0)
    kb = pl.program_id(1)

    @pl.when((bq == 0) & (kb == 0))
    def _():
        out_ref[...] = jnp.zeros_like(out_ref)

    w_chunk = w_ref[0]
    row = jax.lax.broadcasted_iota(jnp.int32, (nb_tc, w_chunk.shape[-1]), 0)
    lhs = jnp.where(row == bq, jnp.broadcast_to(w_chunk, row.shape), 0.0)
    f = feat_ref[0, 0]
    out_ref[...] += jnp.dot(lhs, f, preferred_element_type=jnp.float32)


def _splat(wbuf, tok16, lane):
    wv16 = wbuf[pl.ds(tok16, 16)]
    return lax.gather(
        wv16, lane[:, None],
        lax.GatherDimensionNumbers(
            offset_dims=(), collapsed_slice_dims=(0,), start_index_map=(0,)),
        slice_sizes=(1,),
        mode=lax.GatherScatterMode.PROMISE_IN_BOUNDS)


def _sc_body(feat_hbm, w_hbm, out_hbm, buf0, buf1, wbuf, acc, red, shared,
             sem0, sem1):
    D = 1024
    N = feat_hbm.shape[1]
    c = lax.axis_index("c")
    s = lax.axis_index("s")
    r = 0
    t0 = (c * 16 + s) * _TPT

    pltpu.sync_copy(w_hbm.at[r, 0, pl.ds(pl.multiple_of(t0, _TPT), _TPT)],
                    wbuf)

    bufs = (buf0, buf1)
    sems = (sem0, sem1)
    nch = _TPT // _CH
    handles = [None, None]

    def issue(ci, which):
        tok = pl.multiple_of(t0 + ci * _CH, _CH)
        return pltpu.async_copy(
            feat_hbm.at[r, N - 1, pl.ds(tok, _CH), :], bufs[which],
            sems[which])

    handles[0] = issue(0, 0)

    for ci in range(nch):
        cur = ci % 2
        if ci + 1 < nch:
            handles[(ci + 1) % 2] = issue(ci + 1, (ci + 1) % 2)
        handles[cur].wait()
        cbuf = bufs[cur]

        for g in range(4):
            col0 = g * 256
            if ci == 0:
                init = tuple(jnp.zeros((16,), jnp.float32)
                             for _ in range(16))
            else:
                init = tuple(acc[pl.ds(col0 + j * 16, 16)]
                             for j in range(16))

            def tok_body(i, accs, cbuf=cbuf, ci=ci, col0=col0):
                grp = pl.multiple_of((ci * _CH + i) // 16 * 16, 16)
                lane = jnp.zeros((16,), jnp.int32) + i % 16
                wv = _splat(wbuf, grp, lane)
                return tuple(
                    a + wv * cbuf[i, pl.ds(col0 + j * 16, 16)]
                    for j, a in enumerate(accs))

            accs = lax.fori_loop(0, _CH, tok_body, init)
            for j in range(16):
                acc[pl.ds(col0 + j * 16, 16)] = accs[j]

    pltpu.sync_copy(acc, shared.at[pl.ds(pl.multiple_of(s * D, D), D)])
    plsc.subcore_barrier()

    @pl.when(s == 0)
    def _():
        pltpu.sync_copy(shared, red)
        for db in range(D // 16):
            tot = red[pl.ds(db * 16, 16)]
            for j in range(1, 16):
                tot = tot + red[pl.ds(j * D + db * 16, 16)]
            acc[pl.ds(db * 16, 16)] = tot
        pltpu.sync_copy(acc, out_hbm.at[pl.ds(pl.multiple_of(c * D, D), D)])


def kernel(input_feature, input_lengths, vq_indices):
    B, N, L, D = input_feature.shape
    c0 = vq_indices[:, :, 0].reshape(B, 1, L).astype(jnp.int32)
    c1 = vq_indices[:, :, 1].reshape(B, 1, L).astype(jnp.int32)
    lengths = input_lengths.astype(jnp.int32)

    w = pl.pallas_call(
        _weights_body,
        grid=(B,),
        in_specs=[
            pl.BlockSpec(memory_space=pltpu.SMEM),
            pl.BlockSpec((1, 1, L), lambda b: (b, 0, 0)),
            pl.BlockSpec((1, 1, L), lambda b: (b, 0, 0)),
        ],
        out_specs=pl.BlockSpec((1, 1, L), lambda b: (b, 0, 0)),
        out_shape=jax.ShapeDtypeStruct((B, 1, L), jnp.float32),
    )(lengths, c0, c1)

    mesh = plsc.VectorSubcoreMesh(core_axis_name="c", subcore_axis_name="s")
    sc_call = functools.partial(
        pl.kernel,
        mesh=mesh,
        out_type=jax.ShapeDtypeStruct((2 * D,), jnp.float32),
        scratch_types=[
            pltpu.VMEM((_CH, D), jnp.float32),
            pltpu.VMEM((_CH, D), jnp.float32),
            pltpu.VMEM((_TPT,), jnp.float32),
            pltpu.VMEM((D,), jnp.float32),
            pltpu.VMEM((16 * D,), jnp.float32),
            pltpu.VMEM_SHARED((16 * D,), jnp.float32),
            pltpu.SemaphoreType.DMA,
            pltpu.SemaphoreType.DMA,
        ],
    )(_sc_body)
    parts = sc_call(input_feature, w).reshape(2, D)
    out_sc = (parts[0:1] + parts[1:2])

    nb_tc = B - _NB_SC
    nk = L // _KB
    out_tc = pl.pallas_call(
        _matmul_body,
        grid=(nb_tc, nk),
        in_specs=[
            pl.BlockSpec((1, 1, _KB), lambda bq, kb: (_NB_SC + bq, 0, kb)),
            pl.BlockSpec((1, 1, _KB, D), lambda bq, kb: (_NB_SC + bq, N - 1, kb, 0)),
        ],
        out_specs=pl.BlockSpec((nb_tc, D), lambda bq, kb: (0, 0)),
        out_shape=jax.ShapeDtypeStruct((nb_tc, D), jnp.float32),
    )(w, input_feature)

    return jnp.concatenate([out_sc, out_tc], axis=0)


# speedup vs baseline: 9.3917x; 1.6260x over previous
import jax
import jax.numpy as jnp
from jax.experimental import pallas as pl
from jax.experimental.pallas import tpu as pltpu

_KB = 1024


def _fused_kernel(len_ref, c0_ref, c1_ref, feat_ref, out_ref, w_ref):
    L = c0_ref.shape[-1]
    B = out_ref.shape[0]
    KB = feat_ref.shape[2]
    b = pl.program_id(0)
    kb = pl.program_id(1)

    @pl.when((b == 0) & (kb == 0))
    def _():
        out_ref[...] = jnp.zeros_like(out_ref)

    @pl.when(kb == 0)
    def _():
        n = len_ref[b]
        c0 = c0_ref[0]
        c1 = c1_ref[0]
        pos = jax.lax.broadcasted_iota(jnp.int32, (1, L), 1)
        valid = pos < n

        p0 = jnp.roll(c0, 1, axis=1)
        p1 = jnp.roll(c1, 1, axis=1)
        same = (c0 == p0) & (c1 == p1)
        nb = ((pos == 0) | jnp.logical_not(same)) & valid

        s = jnp.where(nb, pos, -1)
        k = 1
        while k < L:
            sh = jnp.where(pos >= k, jnp.roll(s, k, axis=1), -1)
            s = jnp.maximum(s, sh)
            k *= 2

        big = jnp.int32(2**30)
        t = jnp.where(nb, pos, big)
        k = 1
        while k < L:
            sh = jnp.where(pos < L - k, jnp.roll(t, -k, axis=1), big)
            t = jnp.minimum(t, sh)
            k *= 2
        ns = jnp.where(pos < L - 1, jnp.roll(t, -1, axis=1), big)
        ns = jnp.minimum(ns, n)

        run_len = (ns - s).astype(jnp.float32)
        num_groups = jnp.sum(nb.astype(jnp.float32))
        denom = num_groups * run_len
        safe = valid & (denom > 0)
        w_ref[...] = jnp.where(safe, 1.0 / jnp.where(denom > 0, denom, 1.0), 0.0)

    w_chunk = w_ref[:, pl.ds(kb * KB, KB)]
    row = jax.lax.broadcasted_iota(jnp.int32, (B, KB), 0)
    w_rows = jnp.where(row == b, jnp.broadcast_to(w_chunk, (B, KB)), 0.0)
    f = feat_ref[0, 0]
    out_ref[...] += jnp.dot(w_rows, f, preferred_element_type=jnp.float32)


def kernel(input_feature, input_lengths, vq_indices):
    B, N, L, D = input_feature.shape
    c0 = vq_indices[:, :, 0].reshape(B, 1, L).astype(jnp.int32)
    c1 = vq_indices[:, :, 1].reshape(B, 1, L).astype(jnp.int32)
    lengths = input_lengths.astype(jnp.int32)
    nk = L // _KB

    out = pl.pallas_call(
        _fused_kernel,
        grid=(B, nk),
        in_specs=[
            pl.BlockSpec(memory_space=pltpu.SMEM),
            pl.BlockSpec((1, 1, L), lambda b, kb: (b, 0, 0)),
            pl.BlockSpec((1, 1, L), lambda b, kb: (b, 0, 0)),
            pl.BlockSpec((1, 1, _KB, D), lambda b, kb: (b, N - 1, kb, 0)),
        ],
        out_specs=pl.BlockSpec((B, D), lambda b, kb: (0, 0)),
        out_shape=jax.ShapeDtypeStruct((B, D), jnp.float32),
        scratch_shapes=[pltpu.VMEM((1, L), jnp.float32)],
    )(lengths, c0, c1, input_feature)
    return out
